# Initial kernel scaffold; baseline (speedup 1.0000x reference)
import jax
import jax.numpy as jnp
from jax.experimental import pallas as pl


def kernel(feat_index, weight):
    raise NotImplementedError("write your pallas kernel here")


# speedup vs baseline: 1.5749x; 1.5749x over previous
import functools

import jax
import jax.numpy as jnp
from jax import lax
from jax.experimental import pallas as pl
from jax.experimental.pallas import tpu as pltpu
from jax.experimental.pallas import tpu_sc as plsc

BATCH = 16384
FIELDS = 26
NUM_EMBED = 32

_B = BATCH * FIELDS
_NC, _NS = 2, 16
_NW = _NC * _NS
_BPW = _B // _NW
_CH = 1024
_NCH = _BPW // _CH

assert _BPW * _NW == _B and _NCH * _CH == _BPW


def _emb_body(idx_hbm, table_hbm, out_hbm, idx_v, rows, gsems, wsems):
    wid = lax.axis_index("s") * _NC + lax.axis_index("c")
    base = wid * _BPW
    pltpu.sync_copy(idx_hbm.at[pl.ds(base, _BPW)], idx_v)

    def start_gather(c):
        b = c % 2
        return pltpu.async_copy(
            table_hbm.at[idx_v.at[pl.ds(c * _CH, _CH)]], rows[b], gsems[b]
        )

    gather = [None, None]
    write = [None, None]
    gather[0] = start_gather(0)
    for c in range(_NCH):
        b = c % 2
        if c + 1 < _NCH:
            nb = (c + 1) % 2
            if write[nb] is not None:
                write[nb].wait()
                write[nb] = None
            gather[nb] = start_gather(c + 1)
        gather[b].wait()
        write[b] = pltpu.async_copy(
            rows[b], out_hbm.at[pl.ds(base + c * _CH, _CH)], wsems[b]
        )
    for b in range(2):
        if write[b] is not None:
            write[b].wait()


@jax.jit
def _embedding_sc(flat_idx, weight):
    mesh = plsc.VectorSubcoreMesh(core_axis_name="c", subcore_axis_name="s")
    run = functools.partial(
        pl.kernel,
        out_type=jax.ShapeDtypeStruct((_B, NUM_EMBED), jnp.float32),
        mesh=mesh,
        scratch_types=[
            pltpu.VMEM((_BPW,), jnp.int32),
            [pltpu.VMEM((_CH, NUM_EMBED), jnp.float32) for _ in range(2)],
            [pltpu.SemaphoreType.DMA for _ in range(2)],
            [pltpu.SemaphoreType.DMA for _ in range(2)],
        ],
        compiler_params=pltpu.CompilerParams(use_tc_tiling_on_sc=False),
    )(_emb_body)
    return run(flat_idx, weight)


def kernel(feat_index, weight):
    flat_idx = feat_index.reshape(-1).astype(jnp.int32)
    out = _embedding_sc(flat_idx, weight)
    return out.reshape(BATCH, FIELDS, NUM_EMBED)
